# baseline (speedup 1.0000x reference)
import jax
import jax.numpy as jnp
from jax import lax
from jax.experimental import pallas as pl
from jax.experimental.pallas import tpu as pltpu
from jax.experimental.pallas import tpu_sc as plsc

_N = 10000
_N2 = 10240
_E = 320000
_DIN = 128
_HID = 256
_H = 8
_C = 32
_L = 4
_EHAT = _E + _N
_CB = 512
_EPAD = 335872
_NB = 2048
_EB = 4000
_NT = _N2 // 8
_PERM = [v for t in range(16) for v in (t, 16 + t)]


def _proj(xp, ms_ref, md_ref, as_ref, ad_ref, mxs_ref, mxd_ref):
    a_s = jnp.dot(xp, ms_ref[...], preferred_element_type=jnp.float32)
    a_d = jnp.dot(xp, md_ref[...], preferred_element_type=jnp.float32)
    as_ref[...] = a_s
    ad_ref[...] = a_d
    mxs_ref[...] = jnp.max(a_s, axis=0).reshape(1, 1, _H)
    mxd_ref[...] = jnp.max(a_d, axis=0).reshape(1, 1, _H)


def _enc_pre_body(x_ref, w_ref, b_ref, ws_ref, ms_ref, md_ref,
                  h_ref, xp_ref, as_ref, ad_ref, mxs_ref, mxd_ref):
    y = jnp.dot(x_ref[...], w_ref[...], preferred_element_type=jnp.float32)
    h = jnp.maximum(y + b_ref[...], 0.0)
    h_ref[...] = h
    xp = jnp.dot(h, ws_ref[...], preferred_element_type=jnp.float32)
    xp_ref[...] = xp
    _proj(xp, ms_ref, md_ref, as_ref, ad_ref, mxs_ref, mxd_ref)


def _enc_pre(x, w, b, ws, ms, md):
    g = _N2 // _NB
    return pl.pallas_call(
        _enc_pre_body,
        grid=(g,),
        in_specs=[
            pl.BlockSpec((_NB, _DIN), lambda i: (i, 0)),
            pl.BlockSpec((_DIN, _HID), lambda i: (0, 0)),
            pl.BlockSpec((1, _HID), lambda i: (0, 0)),
            pl.BlockSpec((_HID, _HID), lambda i: (0, 0)),
            pl.BlockSpec((_HID, _H), lambda i: (0, 0)),
            pl.BlockSpec((_HID, _H), lambda i: (0, 0)),
        ],
        out_specs=[
            pl.BlockSpec((_NB, _HID), lambda i: (i, 0)),
            pl.BlockSpec((_NB, _HID), lambda i: (i, 0)),
            pl.BlockSpec((_NB, _H), lambda i: (i, 0)),
            pl.BlockSpec((_NB, _H), lambda i: (i, 0)),
            pl.BlockSpec((1, 1, _H), lambda i: (i, 0, 0)),
            pl.BlockSpec((1, 1, _H), lambda i: (i, 0, 0)),
        ],
        out_shape=[
            jax.ShapeDtypeStruct((_N2, _HID), jnp.float32),
            jax.ShapeDtypeStruct((_N2, _HID), jnp.float32),
            jax.ShapeDtypeStruct((_N2, _H), jnp.float32),
            jax.ShapeDtypeStruct((_N2, _H), jnp.float32),
            jax.ShapeDtypeStruct((g, 1, _H), jnp.float32),
            jax.ShapeDtypeStruct((g, 1, _H), jnp.float32),
        ],
    )(x, w, b.reshape(1, _HID), ws, ms, md)


def _edge_enc_body(e_ref, w_ref, b_ref, v_ref, ae_ref, cs_ref, mx_ref):
    t = jnp.dot(e_ref[...], w_ref[...], preferred_element_type=jnp.float32)
    t = jnp.maximum(t + b_ref[...], 0.0)
    ae = jnp.dot(t, v_ref[...], preferred_element_type=jnp.float32)
    ae_ref[...] = ae
    ones = jnp.ones((1, _EB), jnp.float32)
    cs_ref[...] = jnp.dot(ones, t, preferred_element_type=jnp.float32).reshape(1, 1, _HID)
    mx_ref[...] = jnp.max(ae, axis=0).reshape(1, 1, _L * _H)


def _edge_enc(edge_attr, w, b, v):
    g = _E // _EB
    return pl.pallas_call(
        _edge_enc_body,
        grid=(g,),
        in_specs=[
            pl.BlockSpec((_EB, 16), lambda i: (i, 0)),
            pl.BlockSpec((16, _HID), lambda i: (0, 0)),
            pl.BlockSpec((1, _HID), lambda i: (0, 0)),
            pl.BlockSpec((_HID, _L * _H), lambda i: (0, 0)),
        ],
        out_specs=[
            pl.BlockSpec((_EB, _L * _H), lambda i: (i, 0)),
            pl.BlockSpec((1, 1, _HID), lambda i: (i, 0, 0)),
            pl.BlockSpec((1, 1, _L * _H), lambda i: (i, 0, 0)),
        ],
        out_shape=[
            jax.ShapeDtypeStruct((_E, _L * _H), jnp.float32),
            jax.ShapeDtypeStruct((g, 1, _HID), jnp.float32),
            jax.ShapeDtypeStruct((g, 1, _L * _H), jnp.float32),
        ],
    )(edge_attr, w, b.reshape(1, _HID), v)


def _ln_core(msg_ref, res_ref, b_ref, g_ref, lb_ref):
    out = msg_ref[...] + b_ref[...] + res_ref[...]
    mu = jnp.mean(out, axis=1, keepdims=True)
    d = out - mu
    var = jnp.mean(d * d, axis=1, keepdims=True)
    y = d / jnp.sqrt(var + 1e-5) * g_ref[...] + lb_ref[...]
    return jnp.maximum(y, 0.0)


def _post_pre_body(msg_ref, res_ref, b_ref, g_ref, lb_ref, ws_ref, ms_ref, md_ref,
                   h_ref, xp_ref, as_ref, ad_ref, mxs_ref, mxd_ref):
    h = _ln_core(msg_ref, res_ref, b_ref, g_ref, lb_ref)
    h_ref[...] = h
    xp = jnp.dot(h, ws_ref[...], preferred_element_type=jnp.float32)
    xp_ref[...] = xp
    _proj(xp, ms_ref, md_ref, as_ref, ad_ref, mxs_ref, mxd_ref)


def _post_pre(msg, res, b, g, lb, ws, ms, md):
    gr = _N2 // _NB
    return pl.pallas_call(
        _post_pre_body,
        grid=(gr,),
        in_specs=[
            pl.BlockSpec((_NB, _HID), lambda i: (i, 0)),
            pl.BlockSpec((_NB, _HID), lambda i: (i, 0)),
            pl.BlockSpec((1, _HID), lambda i: (0, 0)),
            pl.BlockSpec((1, _HID), lambda i: (0, 0)),
            pl.BlockSpec((1, _HID), lambda i: (0, 0)),
            pl.BlockSpec((_HID, _HID), lambda i: (0, 0)),
            pl.BlockSpec((_HID, _H), lambda i: (0, 0)),
            pl.BlockSpec((_HID, _H), lambda i: (0, 0)),
        ],
        out_specs=[
            pl.BlockSpec((_NB, _HID), lambda i: (i, 0)),
            pl.BlockSpec((_NB, _HID), lambda i: (i, 0)),
            pl.BlockSpec((_NB, _H), lambda i: (i, 0)),
            pl.BlockSpec((_NB, _H), lambda i: (i, 0)),
            pl.BlockSpec((1, 1, _H), lambda i: (i, 0, 0)),
            pl.BlockSpec((1, 1, _H), lambda i: (i, 0, 0)),
        ],
        out_shape=[
            jax.ShapeDtypeStruct((_N2, _HID), jnp.float32),
            jax.ShapeDtypeStruct((_N2, _HID), jnp.float32),
            jax.ShapeDtypeStruct((_N2, _H), jnp.float32),
            jax.ShapeDtypeStruct((_N2, _H), jnp.float32),
            jax.ShapeDtypeStruct((gr, 1, _H), jnp.float32),
            jax.ShapeDtypeStruct((gr, 1, _H), jnp.float32),
        ],
    )(msg, res, b.reshape(1, _HID), g.reshape(1, _HID), lb.reshape(1, _HID),
      ws, ms, md)


def _post_pool_body(msg_ref, res_ref, b_ref, g_ref, lb_ref, pw_ref, pb_ref,
                    h_ref, s_ref):
    h = _ln_core(msg_ref, res_ref, b_ref, g_ref, lb_ref)
    h_ref[...] = h
    sc = jnp.sum(h * pw_ref[...], axis=1, keepdims=True) + pb_ref[...]
    s_ref[...] = 1.0 / (1.0 + jnp.exp(-sc))


def _post_pool(msg, res, b, g, lb, pw, pb):
    return pl.pallas_call(
        _post_pool_body,
        grid=(_N2 // _NB,),
        in_specs=[
            pl.BlockSpec((_NB, _HID), lambda i: (i, 0)),
            pl.BlockSpec((_NB, _HID), lambda i: (i, 0)),
            pl.BlockSpec((1, _HID), lambda i: (0, 0)),
            pl.BlockSpec((1, _HID), lambda i: (0, 0)),
            pl.BlockSpec((1, _HID), lambda i: (0, 0)),
            pl.BlockSpec((1, _HID), lambda i: (0, 0)),
            pl.BlockSpec((1, 1), lambda i: (0, 0)),
        ],
        out_specs=[
            pl.BlockSpec((_NB, _HID), lambda i: (i, 0)),
            pl.BlockSpec((_NB, 1), lambda i: (i, 0)),
        ],
        out_shape=[
            jax.ShapeDtypeStruct((_N2, _HID), jnp.float32),
            jax.ShapeDtypeStruct((_N2, 1), jnp.float32),
        ],
    )(msg, res, b.reshape(1, _HID), g.reshape(1, _HID), lb.reshape(1, _HID),
      pw.reshape(1, _HID), pb.reshape(1, 1))


def _sc_gat_body(srcp_hbm, dstp_hbm, ae_hbm, asrcT_hbm, adstT_hbm, u_hbm,
                 xpT_hbm, msg_hbm,
                 asrc_v, adst_v, den_v, u_v, d2i, aebuf, exb,
                 ibuf2, jbuf2, rows, bfA, bfB, zrows,
                 den_sh, acc_sh, sem, gsem, ssem):
    c = lax.axis_index("c")
    s = lax.axis_index("s")
    hl2 = s // 8
    q8 = s % 8
    eq8 = _EPAD // 8
    base8 = q8 * eq8
    nc2 = eq8 // (2 * _CB)

    def _zr(i, carry):
        zrows[i, pl.ds(0, 16)] = jnp.zeros((16,), jnp.float32)
        zrows[i, pl.ds(16, 16)] = jnp.zeros((16,), jnp.float32)
        return carry
    lax.fori_loop(0, 128, _zr, 0)

    rbase = hl2 * _N2 + q8 * _NT
    nlo = q8 * _NT

    for p in range(2):
        headb = c * 4 + p * 2 + hl2
        band = (p * 2 + hl2) * 32
        hoff = headb * _N2
        loff = hl2 * _N2

        pltpu.sync_copy(asrcT_hbm.at[pl.ds(headb * _N2, _N2)], asrc_v)
        pltpu.sync_copy(adstT_hbm.at[pl.ds(headb * _N2, _N2)], adst_v)
        pltpu.sync_copy(u_hbm.at[pl.ds(headb * 16, 16)], u_v)
        uvec = u_v[...]

        def _zden(i, carry):
            den_v[i // 32, pl.ds((i % 32) * 16, 16)] = jnp.zeros((16,), jnp.float32)
            return carry
        lax.fori_loop(0, 1024, _zden, 0)
        d2i[pl.ds(0, 16)] = lax.iota(jnp.int32, 16) + band
        d2i[pl.ds(16, 16)] = lax.iota(jnp.int32, 16) + (band + 16)

        def _zacc(i, carry):
            pltpu.sync_copy(zrows, acc_sh.at[pl.ds(rbase + i * 128, 128)])
            return carry
        lax.fori_loop(0, _NT // 128, _zacc, 0)
        plsc.subcore_barrier()

        def _scale(so, bf):
            def _sc(g, inner):
                ev = exb[pl.ds(so + g * 16, 16)]
                for t in range(16):
                    j = g * 16 + t
                    e = ev[t]
                    w = bf[j, :]
                    a, b = plsc.unpack(w, format=plsc.PackFormat.INTERLEAVED)
                    rows[j, pl.ds(0, 16)] = a * e
                    rows[j, pl.ds(16, 16)] = b * e
                return inner
            lax.fori_loop(0, _CB // 16, _sc, 0)

        def _chunk(i2, carry):
            off = base8 + i2 * (2 * _CB)
            row0 = off // 128
            d1 = pltpu.async_copy(srcp_hbm.at[headb, pl.ds(row0, 8)], ibuf2, sem)
            d2 = pltpu.async_copy(dstp_hbm.at[hl2, pl.ds(row0, 8)], jbuf2, sem)
            d3 = pltpu.async_copy(ae_hbm.at[pl.ds(headb * _EPAD + off, 2 * _CB)],
                                  aebuf, sem)
            d1.wait()
            d2.wait()
            ga = [pltpu.async_copy(xpT_hbm.at[ibuf2.at[k]],
                                   bfA.at[pl.ds(k * 128, 128)], gsem)
                  for k in range(4)]
            gb = [pltpu.async_copy(xpT_hbm.at[ibuf2.at[4 + k]],
                                   bfB.at[pl.ds(k * 128, 128)], gsem)
                  for k in range(4)]
            d3.wait()

            def _ex(j, inner):
                r = j // 8
                col = (j % 8) * 16
                iv = ibuf2[r, pl.ds(col, 16)]
                jv = jbuf2[r, pl.ds(col, 16)]
                sv = iv - hoff
                dv = jv - loff
                av = plsc.load_gather(asrc_v, [sv])
                bv = plsc.load_gather(adst_v, [dv])
                al = av + bv + aebuf[pl.ds(j * 16, 16)]
                al = jnp.where(al > 0.0, al, al * 0.2)
                e = jnp.exp(al - uvec)
                exb[pl.ds(j * 16, 16)] = e
                plsc.addupdate_scatter(den_v, [dv >> 9, dv & 511], e)
                return inner
            lax.fori_loop(0, (2 * _CB) // 16, _ex, 0)

            for d in ga:
                d.wait()
            _scale(0, bfA)
            sa = [pltpu.async_copy(rows.at[pl.ds(k * 128, 128)],
                                   acc_sh.at[jbuf2.at[k]], ssem, add=True)
                  for k in range(4)]
            for d in gb:
                d.wait()
            for d in sa:
                d.wait()
            _scale(_CB, bfB)
            sb = [pltpu.async_copy(rows.at[pl.ds(k * 128, 128)],
                                   acc_sh.at[jbuf2.at[4 + k]], ssem, add=True)
                  for k in range(4)]
            for d in sb:
                d.wait()
            return carry
        lax.fori_loop(0, nc2, _chunk, 0)

        @pl.when(q8 == 0)
        def _():
            pltpu.sync_copy(den_v, den_sh.at[pl.ds(band, 32)])
        plsc.subcore_barrier()

        @pl.when(q8 != 0)
        def _():
            pltpu.sync_copy(den_v, den_sh.at[d2i], add=True)
        plsc.subcore_barrier()

        pltpu.sync_copy(den_sh.at[pl.ds(band, 32)], den_v)

        for bo in range(0, _NT, _CB):
            bsz = min(_CB, _NT - bo)
            pltpu.sync_copy(acc_sh.at[pl.ds(rbase + bo, bsz)],
                            rows.at[pl.ds(0, bsz)])

            def _div(g, inner):
                n0 = nlo + bo + g * 16
                dv16 = den_v[n0 >> 9, pl.ds(n0 & 511, 16)]
                rec = 1.0 / (dv16 + 1e-16)
                for t in range(16):
                    e = rec[t]
                    rows[g * 16 + t, pl.ds(0, 16)] = rows[g * 16 + t, pl.ds(0, 16)] * e
                    rows[g * 16 + t, pl.ds(16, 16)] = rows[g * 16 + t, pl.ds(16, 16)] * e
                return inner
            lax.fori_loop(0, bsz // 16, _div, 0)
            pltpu.sync_copy(rows.at[pl.ds(0, bsz)],
                            msg_hbm.at[pl.ds(nlo + bo, bsz),
                                       pl.ds(headb * _C, _C)])
        plsc.subcore_barrier()


def _sc_gat(srcp, dstp, aeT_l, asrcT, adstT, u128, xpT):
    mesh = plsc.VectorSubcoreMesh(core_axis_name="c", subcore_axis_name="s",
                                  num_cores=2, num_subcores=16)
    f = pl.kernel(
        _sc_gat_body,
        out_type=jax.ShapeDtypeStruct((_N2, _HID), jnp.float32),
        mesh=mesh,
        compiler_params=pltpu.CompilerParams(needs_layout_passes=False,
                                             use_tc_tiling_on_sc=False),
        scratch_types=[
            pltpu.VMEM((_N2,), jnp.float32),
            pltpu.VMEM((_N2,), jnp.float32),
            pltpu.VMEM((32, 512), jnp.float32),
            pltpu.VMEM((16,), jnp.float32),
            pltpu.VMEM((32,), jnp.int32),
            pltpu.VMEM((2 * _CB,), jnp.float32),
            pltpu.VMEM((2 * _CB,), jnp.float32),
            pltpu.VMEM((8, 128), jnp.int32),
            pltpu.VMEM((8, 128), jnp.int32),
            pltpu.VMEM((_CB, _C), jnp.float32),
            pltpu.VMEM((_CB, _C), jnp.bfloat16),
            pltpu.VMEM((_CB, _C), jnp.bfloat16),
            pltpu.VMEM((128, _C), jnp.float32),
            pltpu.VMEM_SHARED((128, 512), jnp.float32),
            pltpu.VMEM_SHARED((2 * _N2, _C), jnp.float32),
            pltpu.SemaphoreType.DMA,
            pltpu.SemaphoreType.DMA,
            pltpu.SemaphoreType.DMA,
        ],
    )
    return f(srcp, dstp, aeT_l, asrcT, adstT, u128, xpT)


def kernel(x, edge_index, edge_attr, ne_W, ne_b, ee_W, ee_b, Wsrc, att_s,
           att_d, We, att_e, bias, ln_g, ln_b, pool_W, pool_b):
    npad = _EPAD - _EHAT
    loop = jnp.arange(_N, dtype=jnp.int32)
    zpad = jnp.zeros((npad,), jnp.int32)
    src = jnp.concatenate([edge_index[0].astype(jnp.int32), loop, zpad])
    dst = jnp.concatenate([edge_index[1].astype(jnp.int32), loop, zpad])
    srcp = (src[None, :] + (jnp.arange(_H, dtype=jnp.int32) * _N2)[:, None]
            ).reshape(_H, _EPAD // 128, 128)
    dstp = (dst[None, :] + (jnp.arange(2, dtype=jnp.int32) * _N2)[:, None]
            ).reshape(2, _EPAD // 128, 128)

    V = jnp.einsum('lkhc,lhc->klh', We.reshape(_L, _HID, _H, _C),
                   att_e).reshape(_HID, _L * _H)
    eye = jnp.eye(_H, dtype=jnp.float32)
    ams_all = jnp.einsum('lhc,hg->lhcg', att_s, eye).reshape(_L, _HID, _H)
    amd_all = jnp.einsum('lhc,hg->lhcg', att_d, eye).reshape(_L, _HID, _H)

    xpad = jnp.concatenate(
        [x, jnp.zeros((_N2 - _N, _DIN), jnp.float32)], axis=0)

    ae_e, cs_parts, mx_parts = _edge_enc(edge_attr, ee_W, ee_b, V)
    colmean = cs_parts.sum(axis=(0, 1)) / _E
    ae_loop = colmean @ V
    ae_max = jnp.maximum(mx_parts.max(axis=(0, 1)), ae_loop)
    aeT = jnp.concatenate([
        ae_e.T,
        jnp.broadcast_to(ae_loop[:, None], (_L * _H, _N)),
        jnp.full((_L * _H, npad), -1e30, jnp.float32),
    ], axis=1)

    h, xp, a_s, a_d, mxs, mxd = _enc_pre(xpad, ne_W, ne_b, Wsrc[0],
                                         ams_all[0], amd_all[0])

    scores = None
    for l in range(_L):
        u = mxs.max(axis=(0, 1)) + mxd.max(axis=(0, 1)) + ae_max[l * _H:(l + 1) * _H]
        u = jnp.where(u > 0.0, u, 0.2 * u)
        u128 = jnp.broadcast_to(u[:, None], (_H, 16)).reshape(-1)
        xpb = (xp.reshape(_N2, _H, _C)[:, :, _PERM]
               .transpose(1, 0, 2).reshape(_H * _N2, _C).astype(jnp.bfloat16))
        msg = _sc_gat(srcp, dstp,
                      aeT[l * _H:(l + 1) * _H].reshape(-1),
                      a_s.T.reshape(-1), a_d.T.reshape(-1), u128, xpb)
        if l < _L - 1:
            h, xp, a_s, a_d, mxs, mxd = _post_pre(
                msg, h, bias[l], ln_g[l], ln_b[l],
                Wsrc[l + 1], ams_all[l + 1], amd_all[l + 1])
        else:
            h, scores = _post_pool(msg, h, bias[l], ln_g[l], ln_b[l],
                                   pool_W, pool_b)

    return h[:_N], scores[:_N]


# speedup vs baseline: 1.4029x; 1.4029x over previous
import jax
import jax.numpy as jnp
from jax import lax
from jax.experimental import pallas as pl
from jax.experimental.pallas import tpu as pltpu
from jax.experimental.pallas import tpu_sc as plsc

_N = 10000
_N2 = 10240
_E = 320000
_DIN = 128
_HID = 256
_H = 8
_C = 32
_L = 4
_EHAT = _E + _N
_CB = 512
_EPAD = 335872
_NB = 2048
_EB = 4000
_NT = _N2 // 8


def _proj(xp, ms_ref, md_ref, as_ref, ad_ref, mxs_ref, mxd_ref):
    a_s = jnp.dot(xp, ms_ref[...], preferred_element_type=jnp.float32)
    a_d = jnp.dot(xp, md_ref[...], preferred_element_type=jnp.float32)
    as_ref[...] = a_s
    ad_ref[...] = a_d
    mxs_ref[...] = jnp.max(a_s, axis=0).reshape(1, 1, _H)
    mxd_ref[...] = jnp.max(a_d, axis=0).reshape(1, 1, _H)


def _enc_pre_body(x_ref, w_ref, b_ref, ws_ref, ms_ref, md_ref,
                  h_ref, xp_ref, as_ref, ad_ref, mxs_ref, mxd_ref):
    y = jnp.dot(x_ref[...], w_ref[...], preferred_element_type=jnp.float32)
    h = jnp.maximum(y + b_ref[...], 0.0)
    h_ref[...] = h
    xp = jnp.dot(h, ws_ref[...], preferred_element_type=jnp.float32)
    xp_ref[...] = xp
    _proj(xp, ms_ref, md_ref, as_ref, ad_ref, mxs_ref, mxd_ref)


def _enc_pre(x, w, b, ws, ms, md):
    g = _N2 // _NB
    return pl.pallas_call(
        _enc_pre_body,
        grid=(g,),
        in_specs=[
            pl.BlockSpec((_NB, _DIN), lambda i: (i, 0)),
            pl.BlockSpec((_DIN, _HID), lambda i: (0, 0)),
            pl.BlockSpec((1, _HID), lambda i: (0, 0)),
            pl.BlockSpec((_HID, _HID), lambda i: (0, 0)),
            pl.BlockSpec((_HID, _H), lambda i: (0, 0)),
            pl.BlockSpec((_HID, _H), lambda i: (0, 0)),
        ],
        out_specs=[
            pl.BlockSpec((_NB, _HID), lambda i: (i, 0)),
            pl.BlockSpec((_NB, _HID), lambda i: (i, 0)),
            pl.BlockSpec((_NB, _H), lambda i: (i, 0)),
            pl.BlockSpec((_NB, _H), lambda i: (i, 0)),
            pl.BlockSpec((1, 1, _H), lambda i: (i, 0, 0)),
            pl.BlockSpec((1, 1, _H), lambda i: (i, 0, 0)),
        ],
        out_shape=[
            jax.ShapeDtypeStruct((_N2, _HID), jnp.float32),
            jax.ShapeDtypeStruct((_N2, _HID), jnp.float32),
            jax.ShapeDtypeStruct((_N2, _H), jnp.float32),
            jax.ShapeDtypeStruct((_N2, _H), jnp.float32),
            jax.ShapeDtypeStruct((g, 1, _H), jnp.float32),
            jax.ShapeDtypeStruct((g, 1, _H), jnp.float32),
        ],
    )(x, w, b.reshape(1, _HID), ws, ms, md)


def _edge_enc_body(e_ref, w_ref, b_ref, v_ref, ae_ref, cs_ref, mx_ref):
    t = jnp.dot(e_ref[...], w_ref[...], preferred_element_type=jnp.float32)
    t = jnp.maximum(t + b_ref[...], 0.0)
    ae = jnp.dot(t, v_ref[...], preferred_element_type=jnp.float32)
    ae_ref[...] = ae
    ones = jnp.ones((1, _EB), jnp.float32)
    cs_ref[...] = jnp.dot(ones, t, preferred_element_type=jnp.float32).reshape(1, 1, _HID)
    mx_ref[...] = jnp.max(ae, axis=0).reshape(1, 1, _L * _H)


def _edge_enc(edge_attr, w, b, v):
    g = _E // _EB
    return pl.pallas_call(
        _edge_enc_body,
        grid=(g,),
        in_specs=[
            pl.BlockSpec((_EB, 16), lambda i: (i, 0)),
            pl.BlockSpec((16, _HID), lambda i: (0, 0)),
            pl.BlockSpec((1, _HID), lambda i: (0, 0)),
            pl.BlockSpec((_HID, _L * _H), lambda i: (0, 0)),
        ],
        out_specs=[
            pl.BlockSpec((_EB, _L * _H), lambda i: (i, 0)),
            pl.BlockSpec((1, 1, _HID), lambda i: (i, 0, 0)),
            pl.BlockSpec((1, 1, _L * _H), lambda i: (i, 0, 0)),
        ],
        out_shape=[
            jax.ShapeDtypeStruct((_E, _L * _H), jnp.float32),
            jax.ShapeDtypeStruct((g, 1, _HID), jnp.float32),
            jax.ShapeDtypeStruct((g, 1, _L * _H), jnp.float32),
        ],
    )(edge_attr, w, b.reshape(1, _HID), v)


def _ln_core(msg_ref, res_ref, b_ref, g_ref, lb_ref):
    out = msg_ref[...] + b_ref[...] + res_ref[...]
    mu = jnp.mean(out, axis=1, keepdims=True)
    d = out - mu
    var = jnp.mean(d * d, axis=1, keepdims=True)
    y = d / jnp.sqrt(var + 1e-5) * g_ref[...] + lb_ref[...]
    return jnp.maximum(y, 0.0)


def _post_pre_body(msg_ref, res_ref, b_ref, g_ref, lb_ref, ws_ref, ms_ref, md_ref,
                   h_ref, xp_ref, as_ref, ad_ref, mxs_ref, mxd_ref):
    h = _ln_core(msg_ref, res_ref, b_ref, g_ref, lb_ref)
    h_ref[...] = h
    xp = jnp.dot(h, ws_ref[...], preferred_element_type=jnp.float32)
    xp_ref[...] = xp
    _proj(xp, ms_ref, md_ref, as_ref, ad_ref, mxs_ref, mxd_ref)


def _post_pre(msg, res, b, g, lb, ws, ms, md):
    gr = _N2 // _NB
    return pl.pallas_call(
        _post_pre_body,
        grid=(gr,),
        in_specs=[
            pl.BlockSpec((_NB, _HID), lambda i: (i, 0)),
            pl.BlockSpec((_NB, _HID), lambda i: (i, 0)),
            pl.BlockSpec((1, _HID), lambda i: (0, 0)),
            pl.BlockSpec((1, _HID), lambda i: (0, 0)),
            pl.BlockSpec((1, _HID), lambda i: (0, 0)),
            pl.BlockSpec((_HID, _HID), lambda i: (0, 0)),
            pl.BlockSpec((_HID, _H), lambda i: (0, 0)),
            pl.BlockSpec((_HID, _H), lambda i: (0, 0)),
        ],
        out_specs=[
            pl.BlockSpec((_NB, _HID), lambda i: (i, 0)),
            pl.BlockSpec((_NB, _HID), lambda i: (i, 0)),
            pl.BlockSpec((_NB, _H), lambda i: (i, 0)),
            pl.BlockSpec((_NB, _H), lambda i: (i, 0)),
            pl.BlockSpec((1, 1, _H), lambda i: (i, 0, 0)),
            pl.BlockSpec((1, 1, _H), lambda i: (i, 0, 0)),
        ],
        out_shape=[
            jax.ShapeDtypeStruct((_N2, _HID), jnp.float32),
            jax.ShapeDtypeStruct((_N2, _HID), jnp.float32),
            jax.ShapeDtypeStruct((_N2, _H), jnp.float32),
            jax.ShapeDtypeStruct((_N2, _H), jnp.float32),
            jax.ShapeDtypeStruct((gr, 1, _H), jnp.float32),
            jax.ShapeDtypeStruct((gr, 1, _H), jnp.float32),
        ],
    )(msg, res, b.reshape(1, _HID), g.reshape(1, _HID), lb.reshape(1, _HID),
      ws, ms, md)


def _post_pool_body(msg_ref, res_ref, b_ref, g_ref, lb_ref, pw_ref, pb_ref,
                    h_ref, s_ref):
    h = _ln_core(msg_ref, res_ref, b_ref, g_ref, lb_ref)
    h_ref[...] = h
    sc = jnp.sum(h * pw_ref[...], axis=1, keepdims=True) + pb_ref[...]
    s_ref[...] = 1.0 / (1.0 + jnp.exp(-sc))


def _post_pool(msg, res, b, g, lb, pw, pb):
    return pl.pallas_call(
        _post_pool_body,
        grid=(_N2 // _NB,),
        in_specs=[
            pl.BlockSpec((_NB, _HID), lambda i: (i, 0)),
            pl.BlockSpec((_NB, _HID), lambda i: (i, 0)),
            pl.BlockSpec((1, _HID), lambda i: (0, 0)),
            pl.BlockSpec((1, _HID), lambda i: (0, 0)),
            pl.BlockSpec((1, _HID), lambda i: (0, 0)),
            pl.BlockSpec((1, _HID), lambda i: (0, 0)),
            pl.BlockSpec((1, 1), lambda i: (0, 0)),
        ],
        out_specs=[
            pl.BlockSpec((_NB, _HID), lambda i: (i, 0)),
            pl.BlockSpec((_NB, 1), lambda i: (i, 0)),
        ],
        out_shape=[
            jax.ShapeDtypeStruct((_N2, _HID), jnp.float32),
            jax.ShapeDtypeStruct((_N2, 1), jnp.float32),
        ],
    )(msg, res, b.reshape(1, _HID), g.reshape(1, _HID), lb.reshape(1, _HID),
      pw.reshape(1, _HID), pb.reshape(1, 1))


def _sc_gat_body(srcp_hbm, dstp_hbm, ae_hbm, asrcT_hbm, adstT_hbm, u_hbm,
                 xpT_hbm, msg_hbm,
                 asrc_v, adst_v, den_v, u_v, d2i, aebuf, exb,
                 ibuf2, jbuf2, rows, rowsB, zrows,
                 den_sh, acc_sh, sem, gsem, ssem):
    c = lax.axis_index("c")
    s = lax.axis_index("s")
    hl2 = s // 8
    q8 = s % 8
    eq8 = _EPAD // 8
    base8 = q8 * eq8
    nc2 = eq8 // (2 * _CB)

    def _zr(i, carry):
        zrows[i, pl.ds(0, 16)] = jnp.zeros((16,), jnp.float32)
        zrows[i, pl.ds(16, 16)] = jnp.zeros((16,), jnp.float32)
        return carry
    lax.fori_loop(0, 128, _zr, 0)

    rbase = hl2 * _N2 + q8 * _NT
    nlo = q8 * _NT

    for p in range(2):
        headb = c * 4 + p * 2 + hl2
        band = (p * 2 + hl2) * 32
        hoff = headb * _N2
        loff = hl2 * _N2

        pltpu.sync_copy(asrcT_hbm.at[pl.ds(headb * _N2, _N2)], asrc_v)
        pltpu.sync_copy(adstT_hbm.at[pl.ds(headb * _N2, _N2)], adst_v)
        pltpu.sync_copy(u_hbm.at[pl.ds(headb * 16, 16)], u_v)
        uvec = u_v[...]

        def _zden(i, carry):
            den_v[i // 32, pl.ds((i % 32) * 16, 16)] = jnp.zeros((16,), jnp.float32)
            return carry
        lax.fori_loop(0, 1024, _zden, 0)
        d2i[pl.ds(0, 16)] = lax.iota(jnp.int32, 16) + band
        d2i[pl.ds(16, 16)] = lax.iota(jnp.int32, 16) + (band + 16)

        def _zacc(i, carry):
            pltpu.sync_copy(zrows, acc_sh.at[pl.ds(rbase + i * 128, 128)])
            return carry
        lax.fori_loop(0, _NT // 128, _zacc, 0)
        plsc.subcore_barrier()

        def _scale(so, rw):
            def _sc(g, inner):
                ev = exb[pl.ds(so + g * 16, 16)]
                for t in range(16):
                    j = g * 16 + t
                    e = ev[t]
                    rw[j, pl.ds(0, 16)] = rw[j, pl.ds(0, 16)] * e
                    rw[j, pl.ds(16, 16)] = rw[j, pl.ds(16, 16)] * e
                return inner
            lax.fori_loop(0, _CB // 16, _sc, 0)

        def _chunk(i2, carry):
            off = base8 + i2 * (2 * _CB)
            row0 = off // 128
            d1 = pltpu.async_copy(srcp_hbm.at[headb, pl.ds(row0, 8)], ibuf2, sem)
            d2 = pltpu.async_copy(dstp_hbm.at[hl2, pl.ds(row0, 8)], jbuf2, sem)
            d3 = pltpu.async_copy(ae_hbm.at[pl.ds(headb * _EPAD + off, 2 * _CB)],
                                  aebuf, sem)
            d1.wait()
            d2.wait()
            ga = [pltpu.async_copy(xpT_hbm.at[ibuf2.at[k]],
                                   rows.at[pl.ds(k * 128, 128)], gsem)
                  for k in range(4)]
            gb = [pltpu.async_copy(xpT_hbm.at[ibuf2.at[4 + k]],
                                   rowsB.at[pl.ds(k * 128, 128)], gsem)
                  for k in range(4)]
            d3.wait()

            def _ex(j, inner):
                r = j // 8
                col = (j % 8) * 16
                iv = ibuf2[r, pl.ds(col, 16)]
                jv = jbuf2[r, pl.ds(col, 16)]
                sv = iv - hoff
                dv = jv - loff
                av = plsc.load_gather(asrc_v, [sv])
                bv = plsc.load_gather(adst_v, [dv])
                al = av + bv + aebuf[pl.ds(j * 16, 16)]
                al = jnp.where(al > 0.0, al, al * 0.2)
                e = jnp.exp(al - uvec)
                exb[pl.ds(j * 16, 16)] = e
                plsc.addupdate_scatter(den_v, [dv >> 9, dv & 511], e)
                return inner
            lax.fori_loop(0, (2 * _CB) // 16, _ex, 0)

            for d in ga:
                d.wait()
            _scale(0, rows)
            sa = [pltpu.async_copy(rows.at[pl.ds(k * 128, 128)],
                                   acc_sh.at[jbuf2.at[k]], ssem, add=True)
                  for k in range(4)]
            for d in gb:
                d.wait()
            _scale(_CB, rowsB)
            for d in sa:
                d.wait()
            sb = [pltpu.async_copy(rowsB.at[pl.ds(k * 128, 128)],
                                   acc_sh.at[jbuf2.at[4 + k]], ssem, add=True)
                  for k in range(4)]
            for d in sb:
                d.wait()
            return carry
        lax.fori_loop(0, nc2, _chunk, 0)

        @pl.when(q8 == 0)
        def _():
            pltpu.sync_copy(den_v, den_sh.at[pl.ds(band, 32)])
        plsc.subcore_barrier()

        @pl.when(q8 != 0)
        def _():
            pltpu.sync_copy(den_v, den_sh.at[d2i], add=True)
        plsc.subcore_barrier()

        pltpu.sync_copy(den_sh.at[pl.ds(band, 32)], den_v)

        for bo in range(0, _NT, _CB):
            bsz = min(_CB, _NT - bo)
            pltpu.sync_copy(acc_sh.at[pl.ds(rbase + bo, bsz)],
                            rows.at[pl.ds(0, bsz)])

            def _div(g, inner):
                n0 = nlo + bo + g * 16
                dv16 = den_v[n0 >> 9, pl.ds(n0 & 511, 16)]
                rec = 1.0 / (dv16 + 1e-16)
                for t in range(16):
                    e = rec[t]
                    rows[g * 16 + t, pl.ds(0, 16)] = rows[g * 16 + t, pl.ds(0, 16)] * e
                    rows[g * 16 + t, pl.ds(16, 16)] = rows[g * 16 + t, pl.ds(16, 16)] * e
                return inner
            lax.fori_loop(0, bsz // 16, _div, 0)
            pltpu.sync_copy(rows.at[pl.ds(0, bsz)],
                            msg_hbm.at[pl.ds(nlo + bo, bsz),
                                       pl.ds(headb * _C, _C)])
        plsc.subcore_barrier()


def _sc_gat(srcp, dstp, aeT_l, asrcT, adstT, u128, xpT):
    mesh = plsc.VectorSubcoreMesh(core_axis_name="c", subcore_axis_name="s",
                                  num_cores=2, num_subcores=16)
    f = pl.kernel(
        _sc_gat_body,
        out_type=jax.ShapeDtypeStruct((_N2, _HID), jnp.float32),
        mesh=mesh,
        compiler_params=pltpu.CompilerParams(needs_layout_passes=False,
                                             use_tc_tiling_on_sc=False),
        scratch_types=[
            pltpu.VMEM((_N2,), jnp.float32),
            pltpu.VMEM((_N2,), jnp.float32),
            pltpu.VMEM((32, 512), jnp.float32),
            pltpu.VMEM((16,), jnp.float32),
            pltpu.VMEM((32,), jnp.int32),
            pltpu.VMEM((2 * _CB,), jnp.float32),
            pltpu.VMEM((2 * _CB,), jnp.float32),
            pltpu.VMEM((8, 128), jnp.int32),
            pltpu.VMEM((8, 128), jnp.int32),
            pltpu.VMEM((_CB, _C), jnp.float32),
            pltpu.VMEM((_CB, _C), jnp.float32),
            pltpu.VMEM((128, _C), jnp.float32),
            pltpu.VMEM_SHARED((128, 512), jnp.float32),
            pltpu.VMEM_SHARED((2 * _N2, _C), jnp.float32),
            pltpu.SemaphoreType.DMA,
            pltpu.SemaphoreType.DMA,
            pltpu.SemaphoreType.DMA,
        ],
    )
    return f(srcp, dstp, aeT_l, asrcT, adstT, u128, xpT)


def kernel(x, edge_index, edge_attr, ne_W, ne_b, ee_W, ee_b, Wsrc, att_s,
           att_d, We, att_e, bias, ln_g, ln_b, pool_W, pool_b):
    npad = _EPAD - _EHAT
    loop = jnp.arange(_N, dtype=jnp.int32)
    zpad = jnp.zeros((npad,), jnp.int32)
    src = jnp.concatenate([edge_index[0].astype(jnp.int32), loop, zpad])
    dst = jnp.concatenate([edge_index[1].astype(jnp.int32), loop, zpad])
    srcp = (src[None, :] + (jnp.arange(_H, dtype=jnp.int32) * _N2)[:, None]
            ).reshape(_H, _EPAD // 128, 128)
    dstp = (dst[None, :] + (jnp.arange(2, dtype=jnp.int32) * _N2)[:, None]
            ).reshape(2, _EPAD // 128, 128)

    V = jnp.einsum('lkhc,lhc->klh', We.reshape(_L, _HID, _H, _C),
                   att_e).reshape(_HID, _L * _H)
    eye = jnp.eye(_H, dtype=jnp.float32)
    ams_all = jnp.einsum('lhc,hg->lhcg', att_s, eye).reshape(_L, _HID, _H)
    amd_all = jnp.einsum('lhc,hg->lhcg', att_d, eye).reshape(_L, _HID, _H)

    xpad = jnp.concatenate(
        [x, jnp.zeros((_N2 - _N, _DIN), jnp.float32)], axis=0)

    ae_e, cs_parts, mx_parts = _edge_enc(edge_attr, ee_W, ee_b, V)
    colmean = cs_parts.sum(axis=(0, 1)) / _E
    ae_loop = colmean @ V
    ae_max = jnp.maximum(mx_parts.max(axis=(0, 1)), ae_loop)
    aeT = jnp.concatenate([
        ae_e.T,
        jnp.broadcast_to(ae_loop[:, None], (_L * _H, _N)),
        jnp.full((_L * _H, npad), -1e30, jnp.float32),
    ], axis=1)

    h, xp, a_s, a_d, mxs, mxd = _enc_pre(xpad, ne_W, ne_b, Wsrc[0],
                                         ams_all[0], amd_all[0])

    scores = None
    for l in range(_L):
        u = mxs.max(axis=(0, 1)) + mxd.max(axis=(0, 1)) + ae_max[l * _H:(l + 1) * _H]
        u = jnp.where(u > 0.0, u, 0.2 * u)
        u128 = jnp.broadcast_to(u[:, None], (_H, 16)).reshape(-1)
        xpT = xp.reshape(_N2, _H, _C).transpose(1, 0, 2).reshape(_H * _N2, _C)
        msg = _sc_gat(srcp, dstp,
                      aeT[l * _H:(l + 1) * _H].reshape(-1),
                      a_s.T.reshape(-1), a_d.T.reshape(-1), u128, xpT)
        if l < _L - 1:
            h, xp, a_s, a_d, mxs, mxd = _post_pre(
                msg, h, bias[l], ln_g[l], ln_b[l],
                Wsrc[l + 1], ams_all[l + 1], amd_all[l + 1])
        else:
            h, scores = _post_pool(msg, h, bias[l], ln_g[l], ln_b[l],
                                   pool_W, pool_b)

    return h[:_N], scores[:_N]
